# baseline (speedup 1.0000x reference)
import jax
import jax.numpy as jnp
from jax import lax
from jax.experimental import pallas as pl
from jax.experimental.pallas import tpu as pltpu
from jax.experimental.pallas import tpu_sc as plsc

NUM_CLASSES = 100000
D = 128
K = 20
B = 1024
S = 4
SCALE = float(D) ** 0.5
NC, NS = 2, 16
NW = NC * NS
SEGS = B * S
SEG_PER_W = SEGS // NW
BP_PER_W = SEG_PER_W // S
CHUNKS = BP_PER_W
ROWS = S * K
LANES = 16
NJ = D // LANES
KP = 24


def _bcast16(x, dtype=jnp.int32):
    return jnp.full((LANES,), x, dtype=dtype)


def _bcast_elem(ref, i):
    blk0 = (i // LANES) * LANES
    blk = ref[pl.ds(blk0, LANES)]
    return blk.at[_bcast16(i - blk0)].get(mode="promise_in_bounds")


def _sc_body(selv_hbm, idx_hbm, alpha_hbm, bp_hbm, pres_hbm, valw_hbm,
             query_hbm, pos_hbm, aemb_hbm, tve_hbm,
             outc_hbm, outw_hbm,
             idx_v, selv_v, svacc_v, alpha_v, bp_v, pos_v, aemb_v,
             tve_v, rows_p, rows_v, rows_q, out_c, out_w,
             sem_g, sem_o):
    w = lax.axis_index("s") * NC + lax.axis_index("c")

    pltpu.sync_copy(alpha_hbm.at[pl.ds(w * SEG_PER_W, SEG_PER_W)], alpha_v)
    pltpu.sync_copy(bp_hbm.at[pl.ds(w * BP_PER_W * D, BP_PER_W * D)], bp_v)
    pltpu.sync_copy(pos_hbm, pos_v)
    pltpu.sync_copy(aemb_hbm, aemb_v)
    pltpu.sync_copy(tve_hbm, tve_v)

    saemb = [aemb_v[pl.ds(j * LANES, LANES)] * SCALE for j in range(NJ)]
    stve = [tve_v[pl.ds(j * LANES, LANES)] * SCALE for j in range(NJ)]

    def fetch(c):
        idx_off = w * SEG_PER_W * K + c * ROWS
        pltpu.sync_copy(idx_hbm.at[pl.ds(idx_off, ROWS)], idx_v)
        pltpu.sync_copy(selv_hbm.at[pl.ds(idx_off, ROWS)], selv_v)
        cp1 = pltpu.async_copy(pres_hbm.at[idx_v], rows_p, sem_g)
        cp2 = pltpu.async_copy(valw_hbm.at[idx_v], rows_v, sem_g)
        cp3 = pltpu.async_copy(query_hbm.at[idx_v], rows_q, sem_g)
        cp1.wait()
        cp2.wait()
        cp3.wait()

    def drain_outs(u):
        for g in range(S):
            for o, oh in ((out_c, outc_hbm), (out_w, outw_hbm)):
                pltpu.make_async_copy(
                    o.at[u, pl.ds(g * K * D, K * D)],
                    oh.at[pl.ds(g * K * D, K * D)], sem_o.at[u]).wait()

    def compute(c, u):
        sbase = (w * SEG_PER_W + c * S) * KP * D
        for g in range(S):
            a_b = _bcast_elem(alpha_v, c * S + g)
            cbase = [bp_v[pl.ds(c * D + j * LANES, LANES)] + a_b * saemb[j]
                     for j in range(NJ)]
            zeros = tuple(jnp.zeros((LANES,), jnp.float32) for _ in range(NJ))

            @pl.loop(0, K, init_carry=(zeros, zeros))
            def _kstep(k, carry, g=g, cbase=cbase, u=u):
                accv, accp = carry
                sv = _bcast_elem(selv_v, g * K + k)
                ksl = pl.ds(k * LANES, LANES)
                if g == 0:
                    asv = jnp.zeros((LANES,), jnp.float32)
                    svacc_v[ksl] = sv
                else:
                    asv = svacc_v[ksl]
                    if g < S - 1:
                        svacc_v[ksl] = asv + sv
                r = g * K + k
                naccv, naccp = [], []
                for j in range(NJ):
                    sl = pl.ds(j * LANES, LANES)
                    pres = rows_p[r, sl]
                    vrow = rows_v[r, sl]
                    q = rows_q[r, sl]
                    t = accv[j] + accp[j] + pos_v[pl.ds(k * D + j * LANES,
                                                       LANES)]
                    oc = cbase[j] + asv * stve[j] + t * SCALE
                    ow = oc + (pres + q) * SCALE
                    osl = pl.ds(r * D + j * LANES, LANES)
                    out_c[u, osl] = oc
                    out_w[u, osl] = ow
                    naccv.append(accv[j] + vrow * sv)
                    naccp.append(accp[j] + pres)
                return (tuple(naccv), tuple(naccp))

            dst = pl.ds(sbase + g * KP * D, K * D)
            src = pl.ds(g * K * D, K * D)
            pltpu.async_copy(out_c.at[u, src], outc_hbm.at[dst],
                             sem_o.at[u])
            pltpu.async_copy(out_w.at[u, src], outw_hbm.at[dst],
                             sem_o.at[u])

    @pl.loop(0, CHUNKS, step=2)
    def _chunk(c):
        fetch(c)

        @pl.when(c > 0)
        def _():
            drain_outs(0)

        compute(c, 0)
        fetch(c + 1)

        @pl.when(c > 0)
        def _():
            drain_outs(1)

        compute(c + 1, 1)

    drain_outs(0)
    drain_outs(1)


def kernel(values, indexes, alpha, base_predictor, class_present_w,
           class_value_w, class_query_w, position_embed, alpha_embed,
           tot_values_embed):
    mesh = plsc.VectorSubcoreMesh(core_axis_name="c", subcore_axis_name="s",
                                  num_cores=NC, num_subcores=NS)
    f = pl.kernel(
        _sc_body,
        [jax.ShapeDtypeStruct((SEGS * KP * D,), jnp.float32)] * 2,
        mesh=mesh,
        scratch_types=[
            pltpu.VMEM((ROWS,), jnp.int32),
            pltpu.VMEM((ROWS,), jnp.float32),
            pltpu.VMEM((K * LANES,), jnp.float32),
            pltpu.VMEM((SEG_PER_W,), jnp.float32),
            pltpu.VMEM((BP_PER_W * D,), jnp.float32),
            pltpu.VMEM((K * D,), jnp.float32),
            pltpu.VMEM((D,), jnp.float32),
            pltpu.VMEM((D,), jnp.float32),
            pltpu.VMEM((ROWS, D), jnp.float32),
            pltpu.VMEM((ROWS, D), jnp.float32),
            pltpu.VMEM((ROWS, D), jnp.float32),
            pltpu.VMEM((2, ROWS * D), jnp.float32),
            pltpu.VMEM((2, ROWS * D), jnp.float32),
            pltpu.SemaphoreType.DMA,
            pltpu.SemaphoreType.DMA((2,)),
        ],
    )
    selv = jnp.take_along_axis(values, indexes.reshape(B, S * K), axis=-1)
    oc, ow = f(selv.reshape(-1), indexes.reshape(-1), alpha.reshape(-1),
               base_predictor.reshape(-1), class_present_w, class_value_w,
               class_query_w, position_embed.reshape(-1), alpha_embed,
               tot_values_embed)
    oc = oc.reshape(B, S, KP, D)[:, :, :K, :]
    ow = ow.reshape(B, S, KP, D)[:, :, :K, :]
    return (oc, ow)


# speedup vs baseline: 1.2987x; 1.2987x over previous
import jax
import jax.numpy as jnp
from jax import lax
from jax.experimental import pallas as pl
from jax.experimental.pallas import tpu as pltpu
from jax.experimental.pallas import tpu_sc as plsc

NUM_CLASSES = 100000
D = 128
K = 20
B = 1024
S = 4
SCALE = float(D) ** 0.5
NC, NS = 2, 16
NW = NC * NS
SEGS = B * S
SEG_PER_W = SEGS // NW
BP_PER_W = SEG_PER_W // S
CHUNKS = BP_PER_W
ROWS = S * K
LANES = 16
NJ = D // LANES
KP = 24


def _bcast16(x, dtype=jnp.int32):
    return jnp.full((LANES,), x, dtype=dtype)


def _bcast_elem(ref, i):
    blk0 = (i // LANES) * LANES
    blk = ref[pl.ds(blk0, LANES)]
    return blk.at[_bcast16(i - blk0)].get(mode="promise_in_bounds")


def _sc_body(selv_hbm, idx_hbm, alpha_hbm, bp_hbm, pres_hbm, valw_hbm,
             query_hbm, pos_hbm, aemb_hbm, tve_hbm,
             outc_hbm, outw_hbm,
             idx_v, selv_v, svacc_v, alpha_v, bp_v, pos_v, aemb_v,
             tve_v, rows_p, rows_v, rows_q, out_c, out_w,
             sem_p, sem_v, sem_q):
    w = lax.axis_index("s") * NC + lax.axis_index("c")

    pltpu.sync_copy(alpha_hbm.at[pl.ds(w * SEG_PER_W, SEG_PER_W)], alpha_v)
    pltpu.sync_copy(bp_hbm.at[pl.ds(w * BP_PER_W * D, BP_PER_W * D)], bp_v)
    pltpu.sync_copy(pos_hbm, pos_v)
    pltpu.sync_copy(aemb_hbm, aemb_v)
    pltpu.sync_copy(tve_hbm, tve_v)

    saemb = [aemb_v[pl.ds(j * LANES, LANES)] * SCALE for j in range(NJ)]
    stve = [tve_v[pl.ds(j * LANES, LANES)] * SCALE for j in range(NJ)]

    @pl.loop(0, CHUNKS)
    def _chunk(c):
        idx_off = w * SEG_PER_W * K + c * ROWS

        pltpu.sync_copy(idx_hbm.at[pl.ds(idx_off, ROWS)], idx_v)
        pltpu.sync_copy(selv_hbm.at[pl.ds(idx_off, ROWS)], selv_v)

        cp1 = pltpu.async_copy(pres_hbm.at[idx_v], rows_p, sem_p)
        cp2 = pltpu.async_copy(valw_hbm.at[idx_v], rows_v, sem_v)
        cp3 = pltpu.async_copy(query_hbm.at[idx_v], rows_q, sem_q)
        cp1.wait()
        cp2.wait()
        cp3.wait()

        for g in range(S):
            a_b = _bcast_elem(alpha_v, c * S + g)
            cbase = [bp_v[pl.ds(c * D + j * LANES, LANES)] + a_b * saemb[j]
                     for j in range(NJ)]
            zeros = tuple(jnp.zeros((LANES,), jnp.float32) for _ in range(NJ))

            @pl.loop(0, K, init_carry=(zeros, zeros))
            def _kstep(k, carry, g=g, cbase=cbase):
                accv, accp = carry
                sv = _bcast_elem(selv_v, g * K + k)
                ksl = pl.ds(k * LANES, LANES)
                if g == 0:
                    asv = jnp.zeros((LANES,), jnp.float32)
                    svacc_v[ksl] = sv
                else:
                    asv = svacc_v[ksl]
                    if g < S - 1:
                        svacc_v[ksl] = asv + sv
                r = g * K + k
                naccv, naccp = [], []
                for j in range(NJ):
                    sl = pl.ds(j * LANES, LANES)
                    pres = rows_p[r, sl]
                    vrow = rows_v[r, sl]
                    q = rows_q[r, sl]
                    t = accv[j] + accp[j] + pos_v[pl.ds(k * D + j * LANES,
                                                       LANES)]
                    oc = cbase[j] + asv * stve[j] + t * SCALE
                    ow = oc + (pres + q) * SCALE
                    osl = pl.ds((g * KP + k) * D + j * LANES, LANES)
                    out_c[osl] = oc
                    out_w[osl] = ow
                    naccv.append(accv[j] + vrow * sv)
                    naccp.append(accp[j] + pres)
                return (tuple(naccv), tuple(naccp))

        out_off = (w * SEG_PER_W + c * S) * KP * D
        pltpu.sync_copy(out_c, outc_hbm.at[pl.ds(out_off, S * KP * D)])
        pltpu.sync_copy(out_w, outw_hbm.at[pl.ds(out_off, S * KP * D)])


def kernel(values, indexes, alpha, base_predictor, class_present_w,
           class_value_w, class_query_w, position_embed, alpha_embed,
           tot_values_embed):
    mesh = plsc.VectorSubcoreMesh(core_axis_name="c", subcore_axis_name="s",
                                  num_cores=NC, num_subcores=NS)
    f = pl.kernel(
        _sc_body,
        [jax.ShapeDtypeStruct((SEGS * KP * D,), jnp.float32)] * 2,
        mesh=mesh,
        scratch_types=[
            pltpu.VMEM((ROWS,), jnp.int32),
            pltpu.VMEM((ROWS,), jnp.float32),
            pltpu.VMEM((K * LANES,), jnp.float32),
            pltpu.VMEM((SEG_PER_W,), jnp.float32),
            pltpu.VMEM((BP_PER_W * D,), jnp.float32),
            pltpu.VMEM((K * D,), jnp.float32),
            pltpu.VMEM((D,), jnp.float32),
            pltpu.VMEM((D,), jnp.float32),
            pltpu.VMEM((ROWS, D), jnp.float32),
            pltpu.VMEM((ROWS, D), jnp.float32),
            pltpu.VMEM((ROWS, D), jnp.float32),
            pltpu.VMEM((S * KP * D,), jnp.float32),
            pltpu.VMEM((S * KP * D,), jnp.float32),
            pltpu.SemaphoreType.DMA,
            pltpu.SemaphoreType.DMA,
            pltpu.SemaphoreType.DMA,
        ],
    )
    selv = jnp.take_along_axis(values, indexes.reshape(B, S * K), axis=-1)
    oc, ow = f(selv.reshape(-1), indexes.reshape(-1), alpha.reshape(-1),
               base_predictor.reshape(-1), class_present_w, class_value_w,
               class_query_w, position_embed.reshape(-1), alpha_embed,
               tot_values_embed)
    oc = oc.reshape(B, S, KP, D)[:, :, :K, :]
    ow = ow.reshape(B, S, KP, D)[:, :, :K, :]
    return (oc, ow)
